# baseline (speedup 1.0000x reference)
import functools

import jax
import jax.numpy as jnp
from jax import lax
from jax.experimental import pallas as pl
from jax.experimental.pallas import tpu as pltpu
from jax.experimental.pallas import tpu_sc as plsc

_VOCAB, _DIM, _B, _L = 32320, 1024, 1024, 50
_LF = 48
_NC, _NS = 2, 16
_NW = _NC * _NS
_PER_W = _B // _NW
_TPW = 2 * _PER_W
_TC = 8
_TR = _TPW // _TC

_mesh = plsc.VectorSubcoreMesh(core_axis_name="c", subcore_axis_name="s")


@functools.partial(
    pl.kernel,
    mesh=_mesh,
    out_type=(
        jax.ShapeDtypeStruct((_B, _L, _DIM), jnp.float32),
        jax.ShapeDtypeStruct((_NW, _TPW, _DIM), jnp.float32),
    ),
    scratch_types=[
        pltpu.VMEM((_PER_W, _L), jnp.int32),
        pltpu.VMEM((_TPW,), jnp.int32),
        pltpu.VMEM((_L, _DIM), jnp.float32),
        pltpu.VMEM((_L, _DIM), jnp.float32),
        pltpu.SemaphoreType.DMA,
        pltpu.SemaphoreType.DMA,
    ],
)
def _embed_gather(idx_hbm, tidx_hbm, table_hbm, out_hbm, tail_hbm,
                  idx_v, tidx_v, buf0, buf1, sem0, sem1):
    wid = lax.axis_index("s") * _NC + lax.axis_index("c")
    base = wid * _PER_W
    pltpu.sync_copy(idx_hbm.at[wid], idx_v)
    pltpu.sync_copy(tidx_hbm.at[wid], tidx_v)

    pltpu.async_copy(table_hbm.at[idx_v.at[0]], buf0, sem0)
    pltpu.async_copy(table_hbm.at[idx_v.at[1]], buf1, sem1)

    def body(i, carry):
        r0 = 2 * i
        pltpu.make_async_copy(table_hbm.at[idx_v.at[r0]], buf0, sem0).wait()
        pltpu.sync_copy(buf0.at[pl.ds(0, _LF)], out_hbm.at[base + r0, pl.ds(0, _LF)])
        pltpu.async_copy(table_hbm.at[idx_v.at[r0 + 2]], buf0, sem0)
        r1 = r0 + 1
        pltpu.make_async_copy(table_hbm.at[idx_v.at[r1]], buf1, sem1).wait()
        pltpu.sync_copy(buf1.at[pl.ds(0, _LF)], out_hbm.at[base + r1, pl.ds(0, _LF)])
        pltpu.async_copy(table_hbm.at[idx_v.at[r1 + 2]], buf1, sem1)
        return carry

    lax.fori_loop(0, _PER_W // 2 - 1, body, 0)

    r0 = _PER_W - 2
    pltpu.make_async_copy(table_hbm.at[idx_v.at[r0]], buf0, sem0).wait()
    pltpu.sync_copy(buf0.at[pl.ds(0, _LF)], out_hbm.at[base + r0, pl.ds(0, _LF)])
    r1 = _PER_W - 1
    pltpu.make_async_copy(table_hbm.at[idx_v.at[r1]], buf1, sem1).wait()
    pltpu.sync_copy(buf1.at[pl.ds(0, _LF)], out_hbm.at[base + r1, pl.ds(0, _LF)])

    tb0 = buf0.at[pl.ds(0, _TC)]
    tb1 = buf1.at[pl.ds(0, _TC)]
    pltpu.async_copy(table_hbm.at[tidx_v.at[pl.ds(0, _TC)]], tb0, sem0)
    pltpu.async_copy(table_hbm.at[tidx_v.at[pl.ds(_TC, _TC)]], tb1, sem1)
    for g in range(_TR):
        tb, sem = (tb0, sem0) if g % 2 == 0 else (tb1, sem1)
        pltpu.make_async_copy(
            table_hbm.at[tidx_v.at[pl.ds(g * _TC, _TC)]], tb, sem).wait()
        pltpu.sync_copy(tb, tail_hbm.at[wid, pl.ds(g * _TC, _TC)])
        if g + 2 < _TR:
            pltpu.async_copy(
                table_hbm.at[tidx_v.at[pl.ds((g + 2) * _TC, _TC)]], tb, sem)


def _fixup_body(main_ref, tail_ref, out_ref):
    out_ref[:, 0:2, :] = tail_ref[0].reshape(_PER_W, 2, _DIM)


_fixup = pl.pallas_call(
    _fixup_body,
    grid=(_NW,),
    in_specs=[
        pl.BlockSpec(memory_space=pl.ANY),
        pl.BlockSpec((1, _TPW, _DIM), lambda w: (w, 0, 0)),
    ],
    out_specs=pl.BlockSpec((_PER_W, 8, _DIM), lambda w: (w, 6, 0)),
    out_shape=jax.ShapeDtypeStruct((_B, _L, _DIM), jnp.float32),
    input_output_aliases={0: 0},
)


def kernel(src, src_length, tgt_input, embed_weight):
    idx = src.reshape(_NW, _PER_W, _L)
    tidx = src[:, _LF:_L].reshape(_NW, _TPW)
    out, tail = _embed_gather(idx, tidx, embed_weight)
    out = _fixup(out, tail)
    return out, src_length, tgt_input
